# baseline (speedup 1.0000x reference)
import functools

import jax
import jax.numpy as jnp
from jax import lax
from jax.experimental import pallas as pl
from jax.experimental.pallas import tpu as pltpu
from jax.experimental.pallas import tpu_sc as plsc

B = 8192
INPUT_DIM = 768
LATENT_DIM = 64
NUM_EMB = 8192

BT = 512
NB = B // BT
CHUNK = 2048
NCHUNK = NUM_EMB // CHUNK


def _dot(a, b, dims):
    return lax.dot_general(a, b, (dims, ((), ())),
                           preferred_element_type=jnp.float32)


def _n2_body(cb_ref, n2_ref):
    cb = cb_ref[...]
    n2_ref[...] = jnp.sum(cb * cb, axis=1).reshape(1, NUM_EMB)


def _enc_vq_body(x_ref, W1_ref, b1_ref, W2_ref, b2_ref, W3_ref, b3_ref,
                 cb_ref, n2_ref, z_ref, idx_ref):
    x = x_ref[...]
    h = jnp.maximum(_dot(x, W1_ref[...], ((1,), (0,))) + b1_ref[...], 0.0)
    h = jnp.maximum(_dot(h, W2_ref[...], ((1,), (0,))) + b2_ref[...], 0.0)
    z = _dot(h, W3_ref[...], ((1,), (0,))) + b3_ref[...]
    z_ref[...] = z

    zz = jnp.sum(z * z, axis=1, keepdims=True)
    z2 = z + z
    best = jnp.full((BT,), jnp.inf, dtype=jnp.float32)
    besti = jnp.zeros((BT,), dtype=jnp.int32)
    for j in range(NCHUNK):
        cb = cb_ref[j * CHUNK:(j + 1) * CHUNK, :]
        n2 = n2_ref[0:1, j * CHUNK:(j + 1) * CHUNK]
        d = zz - _dot(z2, cb, ((1,), (1,))) + n2
        lmin = jnp.min(d, axis=1)
        col = lax.broadcasted_iota(jnp.int32, (BT, CHUNK), 1)
        lidx = jnp.min(jnp.where(d == lmin[:, None], col, NUM_EMB), axis=1)
        upd = lmin < best
        best = jnp.where(upd, lmin, best)
        besti = jnp.where(upd, lidx + j * CHUNK, besti)
    idx_ref[0, 0, :] = besti


def _dec_body(z_ref, zq_ref, D1_ref, c1_ref, D2_ref, c2_ref, D3_ref, c3_ref,
              xr_ref, loss_ref):
    z = z_ref[...]
    zq = zq_ref[:, :LATENT_DIM]
    zst = z + (zq - z)
    h = jnp.maximum(_dot(zst, D1_ref[...], ((1,), (0,))) + c1_ref[...], 0.0)
    h = jnp.maximum(_dot(h, D2_ref[...], ((1,), (0,))) + c2_ref[...], 0.0)
    xr_ref[...] = jnp.tanh(_dot(h, D3_ref[...], ((1,), (0,))) + c3_ref[...])

    part = jnp.sum((zq - z) ** 2).reshape(1, 1)

    @pl.when(pl.program_id(0) == 0)
    def _init():
        loss_ref[...] = part

    @pl.when(pl.program_id(0) != 0)
    def _acc():
        loss_ref[...] += part


def _const_spec(shape):
    return pl.BlockSpec(shape, lambda i: (0,) * len(shape))


GD = 128


def _sc_gather(codebook_padded, idx):
    n = idx.shape[0]
    info = plsc.get_sparse_core_info()
    nw = info.num_cores * info.num_subcores
    bpw = n // nw
    mesh = plsc.VectorSubcoreMesh(core_axis_name="c", subcore_axis_name="s")

    nseg = 4
    seg = bpw // nseg

    @functools.partial(
        pl.kernel, mesh=mesh,
        out_type=jax.ShapeDtypeStruct((n, GD), jnp.float32),
        scratch_types=[
            pltpu.VMEM((nseg, seg), jnp.int32),
            pltpu.VMEM((bpw, GD), jnp.float32),
            pltpu.SemaphoreType.DMA,
        ],
    )
    def gather_k(table_hbm, idx_hbm, out_hbm, idx_v, rows_v, sem):
        wid = lax.axis_index("s") * info.num_cores + lax.axis_index("c")
        base = wid * bpw
        for k in range(nseg):
            pltpu.sync_copy(idx_hbm.at[pl.ds(base + k * seg, seg)], idx_v.at[k])
        copies = [
            pltpu.async_copy(table_hbm.at[idx_v.at[k]],
                             rows_v.at[pl.ds(k * seg, seg)], sem)
            for k in range(nseg)
        ]
        for c in copies:
            c.wait()
        pltpu.sync_copy(rows_v, out_hbm.at[pl.ds(base, bpw)])

    return gather_k(codebook_padded, idx)


def kernel(x, W1, b1, W2, b2, W3, b3, codebook, D1, c1, D2, c2, D3, c3):
    n2 = pl.pallas_call(
        _n2_body,
        in_specs=[pl.BlockSpec((NUM_EMB, LATENT_DIM), lambda: (0, 0))],
        out_specs=pl.BlockSpec((1, NUM_EMB), lambda: (0, 0)),
        out_shape=jax.ShapeDtypeStruct((1, NUM_EMB), jnp.float32),
    )(codebook)

    H = B // 2
    NBH = H // BT

    def stage_a(xh):
        return pl.pallas_call(
            _enc_vq_body,
            grid=(NBH,),
            in_specs=[
                pl.BlockSpec((BT, INPUT_DIM), lambda i: (i, 0)),
                _const_spec((INPUT_DIM, 512)),
                _const_spec((1, 512)),
                _const_spec((512, 256)),
                _const_spec((1, 256)),
                _const_spec((256, LATENT_DIM)),
                _const_spec((1, LATENT_DIM)),
                _const_spec((NUM_EMB, LATENT_DIM)),
                _const_spec((1, NUM_EMB)),
            ],
            out_specs=[
                pl.BlockSpec((BT, LATENT_DIM), lambda i: (i, 0)),
                pl.BlockSpec((1, 1, BT), lambda i: (i, 0, 0)),
            ],
            out_shape=[
                jax.ShapeDtypeStruct((H, LATENT_DIM), jnp.float32),
                jax.ShapeDtypeStruct((NBH, 1, BT), jnp.int32),
            ],
        )(xh, W1, b1.reshape(1, -1), W2, b2.reshape(1, -1), W3,
          b3.reshape(1, -1), codebook, n2)

    cb_pad = jnp.pad(codebook, ((0, 0), (0, GD - LATENT_DIM)))
    z0, i0 = stage_a(x[:H])
    idx0 = i0.reshape(H)
    zq0 = _sc_gather(cb_pad, idx0)
    z1, i1 = stage_a(x[H:])
    idx1 = i1.reshape(H)
    zq1 = _sc_gather(cb_pad, idx1)

    z = jnp.concatenate([z0, z1], axis=0)
    idx = jnp.concatenate([idx0, idx1], axis=0)
    zq = jnp.concatenate([zq0, zq1], axis=0)

    xr, loss = pl.pallas_call(
        _dec_body,
        grid=(NB,),
        in_specs=[
            pl.BlockSpec((BT, LATENT_DIM), lambda i: (i, 0)),
            pl.BlockSpec((BT, GD), lambda i: (i, 0)),
            _const_spec((LATENT_DIM, 256)),
            _const_spec((1, 256)),
            _const_spec((256, 512)),
            _const_spec((1, 512)),
            _const_spec((512, INPUT_DIM)),
            _const_spec((1, INPUT_DIM)),
        ],
        out_specs=[
            pl.BlockSpec((BT, INPUT_DIM), lambda i: (i, 0)),
            _const_spec((1, 1)),
        ],
        out_shape=[
            jax.ShapeDtypeStruct((B, INPUT_DIM), jnp.float32),
            jax.ShapeDtypeStruct((1, 1), jnp.float32),
        ],
    )(z, zq, D1, c1.reshape(1, -1), D2, c2.reshape(1, -1), D3,
      c3.reshape(1, -1))

    commitment_loss = 0.25 * (loss[0, 0] / (B * LATENT_DIM))
    return (xr, z, idx, commitment_loss)


# speedup vs baseline: 1.1022x; 1.1022x over previous
import functools

import jax
import jax.numpy as jnp
from jax import lax
from jax.experimental import pallas as pl
from jax.experimental.pallas import tpu as pltpu
from jax.experimental.pallas import tpu_sc as plsc

B = 8192
INPUT_DIM = 768
LATENT_DIM = 64
NUM_EMB = 8192

BT = 512
NB = B // BT
CHUNK = 2048
NCHUNK = NUM_EMB // CHUNK


def _dot(a, b, dims):
    return lax.dot_general(a, b, (dims, ((), ())),
                           preferred_element_type=jnp.float32)


def _n2_body(cb_ref, n2_ref):
    cb = cb_ref[...]
    n2_ref[...] = jnp.sum(cb * cb, axis=1).reshape(1, NUM_EMB)


def _enc_vq_body(x_ref, W1_ref, b1_ref, W2_ref, b2_ref, W3_ref, b3_ref,
                 cb_ref, n2_ref, z_ref, idx_ref):
    x = x_ref[...]
    h = jnp.maximum(_dot(x, W1_ref[...], ((1,), (0,))) + b1_ref[...], 0.0)
    h = jnp.maximum(_dot(h, W2_ref[...], ((1,), (0,))) + b2_ref[...], 0.0)
    z = _dot(h, W3_ref[...], ((1,), (0,))) + b3_ref[...]
    z_ref[...] = z

    zz = jnp.sum(z * z, axis=1, keepdims=True)
    z2 = z + z
    best = jnp.full((BT,), jnp.inf, dtype=jnp.float32)
    besti = jnp.zeros((BT,), dtype=jnp.int32)
    for j in range(NCHUNK):
        cb = cb_ref[j * CHUNK:(j + 1) * CHUNK, :]
        n2 = n2_ref[0:1, j * CHUNK:(j + 1) * CHUNK]
        d = zz - _dot(z2, cb, ((1,), (1,))) + n2
        lmin = jnp.min(d, axis=1)
        col = lax.broadcasted_iota(jnp.int32, (BT, CHUNK), 1)
        lidx = jnp.min(jnp.where(d == lmin[:, None], col, NUM_EMB), axis=1)
        upd = lmin < best
        best = jnp.where(upd, lmin, best)
        besti = jnp.where(upd, lidx + j * CHUNK, besti)
    idx_ref[0, 0, :] = besti


def _dec_body(z_ref, zq_ref, D1_ref, c1_ref, D2_ref, c2_ref, D3_ref, c3_ref,
              xr_ref, loss_ref):
    z = z_ref[...]
    zq = zq_ref[:, :LATENT_DIM]
    zst = z + (zq - z)
    h = jnp.maximum(_dot(zst, D1_ref[...], ((1,), (0,))) + c1_ref[...], 0.0)
    h = jnp.maximum(_dot(h, D2_ref[...], ((1,), (0,))) + c2_ref[...], 0.0)
    xr_ref[...] = jnp.tanh(_dot(h, D3_ref[...], ((1,), (0,))) + c3_ref[...])

    part = jnp.sum((zq - z) ** 2).reshape(1, 1)

    @pl.when(pl.program_id(0) == 0)
    def _init():
        loss_ref[...] = part

    @pl.when(pl.program_id(0) != 0)
    def _acc():
        loss_ref[...] += part


def _const_spec(shape):
    return pl.BlockSpec(shape, lambda i: (0,) * len(shape))


GD = 128


def _sc_gather(codebook_padded, idx):
    n = idx.shape[0]
    info = plsc.get_sparse_core_info()
    ns = info.num_subcores
    nw = info.num_cores * ns
    bpw = n // nw
    rows_per_tile = NUM_EMB // ns
    mesh = plsc.VectorSubcoreMesh(core_axis_name="c", subcore_axis_name="s")

    @functools.partial(
        pl.kernel, mesh=mesh,
        out_type=jax.ShapeDtypeStruct((n, GD), jnp.float32),
        scratch_types=[
            pltpu.VMEM((bpw,), jnp.int32),
            pltpu.VMEM((bpw, GD), jnp.float32),
            pltpu.VMEM_SHARED((NUM_EMB, GD), jnp.float32),
            pltpu.SemaphoreType.DMA,
        ],
    )
    def gather_k(table_hbm, idx_hbm, out_hbm, idx_v, rows_v, shared, sem):
        cid = lax.axis_index("c")
        sid = lax.axis_index("s")
        wid = sid * info.num_cores + cid
        sbase = sid * rows_per_tile
        pltpu.sync_copy(table_hbm.at[pl.ds(sbase, rows_per_tile)],
                        shared.at[pl.ds(sbase, rows_per_tile)])
        plsc.subcore_barrier()
        base = wid * bpw
        pltpu.sync_copy(idx_hbm.at[pl.ds(base, bpw)], idx_v)
        pltpu.async_copy(shared.at[idx_v], rows_v, sem).wait()
        pltpu.sync_copy(rows_v, out_hbm.at[pl.ds(base, bpw)])

    return gather_k(codebook_padded, idx)


def kernel(x, W1, b1, W2, b2, W3, b3, codebook, D1, c1, D2, c2, D3, c3):
    n2 = pl.pallas_call(
        _n2_body,
        in_specs=[pl.BlockSpec((NUM_EMB, LATENT_DIM), lambda: (0, 0))],
        out_specs=pl.BlockSpec((1, NUM_EMB), lambda: (0, 0)),
        out_shape=jax.ShapeDtypeStruct((1, NUM_EMB), jnp.float32),
    )(codebook)

    H = B // 2
    NBH = H // BT

    def stage_a(xh):
        return pl.pallas_call(
            _enc_vq_body,
            grid=(NBH,),
            in_specs=[
                pl.BlockSpec((BT, INPUT_DIM), lambda i: (i, 0)),
                _const_spec((INPUT_DIM, 512)),
                _const_spec((1, 512)),
                _const_spec((512, 256)),
                _const_spec((1, 256)),
                _const_spec((256, LATENT_DIM)),
                _const_spec((1, LATENT_DIM)),
                _const_spec((NUM_EMB, LATENT_DIM)),
                _const_spec((1, NUM_EMB)),
            ],
            out_specs=[
                pl.BlockSpec((BT, LATENT_DIM), lambda i: (i, 0)),
                pl.BlockSpec((1, 1, BT), lambda i: (i, 0, 0)),
            ],
            out_shape=[
                jax.ShapeDtypeStruct((H, LATENT_DIM), jnp.float32),
                jax.ShapeDtypeStruct((NBH, 1, BT), jnp.int32),
            ],
        )(xh, W1, b1.reshape(1, -1), W2, b2.reshape(1, -1), W3,
          b3.reshape(1, -1), codebook, n2)

    cb_pad = jnp.pad(codebook, ((0, 0), (0, GD - LATENT_DIM)))
    z0, i0 = stage_a(x[:H])
    idx0 = i0.reshape(H)
    zq0 = _sc_gather(cb_pad, idx0)
    z1, i1 = stage_a(x[H:])
    idx1 = i1.reshape(H)
    zq1 = _sc_gather(cb_pad, idx1)

    z = jnp.concatenate([z0, z1], axis=0)
    idx = jnp.concatenate([idx0, idx1], axis=0)
    zq = jnp.concatenate([zq0, zq1], axis=0)

    xr, loss = pl.pallas_call(
        _dec_body,
        grid=(NB,),
        in_specs=[
            pl.BlockSpec((BT, LATENT_DIM), lambda i: (i, 0)),
            pl.BlockSpec((BT, GD), lambda i: (i, 0)),
            _const_spec((LATENT_DIM, 256)),
            _const_spec((1, 256)),
            _const_spec((256, 512)),
            _const_spec((1, 512)),
            _const_spec((512, INPUT_DIM)),
            _const_spec((1, INPUT_DIM)),
        ],
        out_specs=[
            pl.BlockSpec((BT, INPUT_DIM), lambda i: (i, 0)),
            _const_spec((1, 1)),
        ],
        out_shape=[
            jax.ShapeDtypeStruct((B, INPUT_DIM), jnp.float32),
            jax.ShapeDtypeStruct((1, 1), jnp.float32),
        ],
    )(z, zq, D1, c1.reshape(1, -1), D2, c2.reshape(1, -1), D3,
      c3.reshape(1, -1))

    commitment_loss = 0.25 * (loss[0, 0] / (B * LATENT_DIM))
    return (xr, z, idx, commitment_loss)


# speedup vs baseline: 1.2006x; 1.0893x over previous
import functools

import jax
import jax.numpy as jnp
from jax import lax
from jax.experimental import pallas as pl
from jax.experimental.pallas import tpu as pltpu
from jax.experimental.pallas import tpu_sc as plsc

B = 8192
INPUT_DIM = 768
LATENT_DIM = 64
NUM_EMB = 8192

BT = 512
NB = B // BT
CHUNK = 2048
NCHUNK = NUM_EMB // CHUNK


def _dot(a, b, dims):
    return lax.dot_general(a, b, (dims, ((), ())),
                           preferred_element_type=jnp.float32)


def _n2_body(cb_ref, n2_ref):
    cb = cb_ref[...]
    n2_ref[...] = jnp.sum(cb * cb, axis=1).reshape(1, NUM_EMB)


def _enc_vq_body(x_ref, W1_ref, b1_ref, W2_ref, b2_ref, W3_ref, b3_ref,
                 cb_ref, n2_ref, z_ref, idx_ref):
    x = x_ref[...]
    h = jnp.maximum(_dot(x, W1_ref[...], ((1,), (0,))) + b1_ref[...], 0.0)
    h = jnp.maximum(_dot(h, W2_ref[...], ((1,), (0,))) + b2_ref[...], 0.0)
    z = _dot(h, W3_ref[...], ((1,), (0,))) + b3_ref[...]
    z_ref[...] = z

    zz = jnp.sum(z * z, axis=1, keepdims=True)
    z2 = z + z
    best = jnp.full((BT,), jnp.inf, dtype=jnp.float32)
    besti = jnp.zeros((BT,), dtype=jnp.int32)
    for j in range(NCHUNK):
        cb = cb_ref[j * CHUNK:(j + 1) * CHUNK, :]
        n2 = n2_ref[0:1, j * CHUNK:(j + 1) * CHUNK]
        d = zz - _dot(z2, cb, ((1,), (1,))) + n2
        lmin = jnp.min(d, axis=1)
        col = lax.broadcasted_iota(jnp.int32, (BT, CHUNK), 1)
        lidx = jnp.min(jnp.where(d == lmin[:, None], col, NUM_EMB), axis=1)
        upd = lmin < best
        best = jnp.where(upd, lmin, best)
        besti = jnp.where(upd, lidx + j * CHUNK, besti)
    idx_ref[0, 0, :] = besti


def _dec_body(z_ref, zq_ref, D1_ref, c1_ref, D2_ref, c2_ref, D3_ref, c3_ref,
              xr_ref, loss_ref):
    z = z_ref[...]
    zq = zq_ref[:, :LATENT_DIM]
    zst = z + (zq - z)
    h = jnp.maximum(_dot(zst, D1_ref[...], ((1,), (0,))) + c1_ref[...], 0.0)
    h = jnp.maximum(_dot(h, D2_ref[...], ((1,), (0,))) + c2_ref[...], 0.0)
    xr_ref[...] = jnp.tanh(_dot(h, D3_ref[...], ((1,), (0,))) + c3_ref[...])

    part = jnp.sum((zq - z) ** 2).reshape(1, 1)

    @pl.when(pl.program_id(0) == 0)
    def _init():
        loss_ref[...] = part

    @pl.when(pl.program_id(0) != 0)
    def _acc():
        loss_ref[...] += part


def _const_spec(shape):
    return pl.BlockSpec(shape, lambda i: (0,) * len(shape))


GD = 128


def _sc_gather(codebook_padded, idx):
    n = idx.shape[0]
    info = plsc.get_sparse_core_info()
    ns = info.num_subcores
    nw = info.num_cores * ns
    bpw = n // nw
    rows_per_tile = NUM_EMB // ns
    mesh = plsc.VectorSubcoreMesh(core_axis_name="c", subcore_axis_name="s")

    @functools.partial(
        pl.kernel, mesh=mesh,
        out_type=jax.ShapeDtypeStruct((n, GD), jnp.float32),
        scratch_types=[
            pltpu.VMEM((bpw,), jnp.int32),
            pltpu.VMEM((bpw, GD), jnp.float32),
            pltpu.VMEM_SHARED((NUM_EMB, GD), jnp.float32),
            pltpu.SemaphoreType.DMA,
        ],
    )
    def gather_k(table_hbm, idx_hbm, out_hbm, idx_v, rows_v, shared, sem):
        cid = lax.axis_index("c")
        sid = lax.axis_index("s")
        wid = sid * info.num_cores + cid
        sbase = sid * rows_per_tile
        pltpu.sync_copy(table_hbm.at[pl.ds(sbase, rows_per_tile)],
                        shared.at[pl.ds(sbase, rows_per_tile)])
        plsc.subcore_barrier()
        base = wid * bpw
        pltpu.sync_copy(idx_hbm.at[pl.ds(base, bpw)], idx_v)
        pltpu.async_copy(shared.at[idx_v], rows_v, sem).wait()
        pltpu.sync_copy(rows_v, out_hbm.at[pl.ds(base, bpw)])

    return gather_k(codebook_padded, idx)


def kernel(x, W1, b1, W2, b2, W3, b3, codebook, D1, c1, D2, c2, D3, c3):
    n2 = pl.pallas_call(
        _n2_body,
        in_specs=[pl.BlockSpec((NUM_EMB, LATENT_DIM), lambda: (0, 0))],
        out_specs=pl.BlockSpec((1, NUM_EMB), lambda: (0, 0)),
        out_shape=jax.ShapeDtypeStruct((1, NUM_EMB), jnp.float32),
    )(codebook)

    H = B // 2
    NBH = H // BT

    def stage_a(phase):
        off = phase * NBH
        return pl.pallas_call(
            _enc_vq_body,
            grid=(NBH,),
            in_specs=[
                pl.BlockSpec((BT, INPUT_DIM), lambda i: (i + off, 0)),
                _const_spec((INPUT_DIM, 512)),
                _const_spec((1, 512)),
                _const_spec((512, 256)),
                _const_spec((1, 256)),
                _const_spec((256, LATENT_DIM)),
                _const_spec((1, LATENT_DIM)),
                _const_spec((NUM_EMB, LATENT_DIM)),
                _const_spec((1, NUM_EMB)),
            ],
            out_specs=[
                pl.BlockSpec((BT, LATENT_DIM), lambda i: (i, 0)),
                pl.BlockSpec((1, 1, BT), lambda i: (i, 0, 0)),
            ],
            out_shape=[
                jax.ShapeDtypeStruct((H, LATENT_DIM), jnp.float32),
                jax.ShapeDtypeStruct((NBH, 1, BT), jnp.int32),
            ],
        )(x, W1, b1.reshape(1, -1), W2, b2.reshape(1, -1), W3,
          b3.reshape(1, -1), codebook, n2)

    cb_pad = jnp.pad(codebook, ((0, 0), (0, GD - LATENT_DIM)))
    z0, i0 = stage_a(0)
    idx0 = i0.reshape(H)
    zq0 = _sc_gather(cb_pad, idx0)
    z1, i1 = stage_a(1)
    idx1 = i1.reshape(H)
    zq1 = _sc_gather(cb_pad, idx1)

    z = jnp.concatenate([z0, z1], axis=0)
    idx = jnp.concatenate([idx0, idx1], axis=0)
    zq = jnp.concatenate([zq0, zq1], axis=0)

    xr, loss = pl.pallas_call(
        _dec_body,
        grid=(NB,),
        in_specs=[
            pl.BlockSpec((BT, LATENT_DIM), lambda i: (i, 0)),
            pl.BlockSpec((BT, GD), lambda i: (i, 0)),
            _const_spec((LATENT_DIM, 256)),
            _const_spec((1, 256)),
            _const_spec((256, 512)),
            _const_spec((1, 512)),
            _const_spec((512, INPUT_DIM)),
            _const_spec((1, INPUT_DIM)),
        ],
        out_specs=[
            pl.BlockSpec((BT, INPUT_DIM), lambda i: (i, 0)),
            _const_spec((1, 1)),
        ],
        out_shape=[
            jax.ShapeDtypeStruct((B, INPUT_DIM), jnp.float32),
            jax.ShapeDtypeStruct((1, 1), jnp.float32),
        ],
    )(z, zq, D1, c1.reshape(1, -1), D2, c2.reshape(1, -1), D3,
      c3.reshape(1, -1))

    commitment_loss = 0.25 * (loss[0, 0] / (B * LATENT_DIM))
    return (xr, z, idx, commitment_loss)


# speedup vs baseline: 1.2133x; 1.0106x over previous
import functools

import jax
import jax.numpy as jnp
from jax import lax
from jax.experimental import pallas as pl
from jax.experimental.pallas import tpu as pltpu
from jax.experimental.pallas import tpu_sc as plsc

B = 8192
INPUT_DIM = 768
LATENT_DIM = 64
NUM_EMB = 8192

BT = 512
NB = B // BT
CHUNK = 2048
NCHUNK = NUM_EMB // CHUNK


def _dot(a, b, dims):
    return lax.dot_general(a, b, (dims, ((), ())),
                           preferred_element_type=jnp.float32)


def _n2_body(cb_ref, n2_ref, cbp_ref):
    cb = cb_ref[...]
    n2_ref[...] = jnp.sum(cb * cb, axis=1).reshape(1, NUM_EMB)
    cbp_ref[:, :LATENT_DIM] = cb
    cbp_ref[:, LATENT_DIM:] = jnp.zeros((NUM_EMB, GD - LATENT_DIM), jnp.float32)


def _enc_vq_body(x_ref, W1_ref, b1_ref, W2_ref, b2_ref, W3_ref, b3_ref,
                 cb_ref, n2_ref, z_ref, idx_ref):
    x = x_ref[...]
    h = jnp.maximum(_dot(x, W1_ref[...], ((1,), (0,))) + b1_ref[...], 0.0)
    h = jnp.maximum(_dot(h, W2_ref[...], ((1,), (0,))) + b2_ref[...], 0.0)
    z = _dot(h, W3_ref[...], ((1,), (0,))) + b3_ref[...]
    z_ref[...] = z

    zz = jnp.sum(z * z, axis=1, keepdims=True)
    z2 = z + z
    best = jnp.full((BT,), jnp.inf, dtype=jnp.float32)
    besti = jnp.zeros((BT,), dtype=jnp.int32)
    for j in range(NCHUNK):
        cb = cb_ref[j * CHUNK:(j + 1) * CHUNK, :]
        n2 = n2_ref[0:1, j * CHUNK:(j + 1) * CHUNK]
        d = zz - _dot(z2, cb, ((1,), (1,))) + n2
        lmin = jnp.min(d, axis=1)
        col = lax.broadcasted_iota(jnp.int32, (BT, CHUNK), 1)
        lidx = jnp.min(jnp.where(d == lmin[:, None], col, NUM_EMB), axis=1)
        upd = lmin < best
        best = jnp.where(upd, lmin, best)
        besti = jnp.where(upd, lidx + j * CHUNK, besti)
    idx_ref[0, 0, :] = besti


def _dec_body(z_ref, zq_ref, D1_ref, c1_ref, D2_ref, c2_ref, D3_ref, c3_ref,
              xr_ref, loss_ref):
    z = z_ref[...]
    zq = zq_ref[:, :LATENT_DIM]
    zst = z + (zq - z)
    h = jnp.maximum(_dot(zst, D1_ref[...], ((1,), (0,))) + c1_ref[...], 0.0)
    h = jnp.maximum(_dot(h, D2_ref[...], ((1,), (0,))) + c2_ref[...], 0.0)
    xr_ref[...] = jnp.tanh(_dot(h, D3_ref[...], ((1,), (0,))) + c3_ref[...])

    part = jnp.sum((zq - z) ** 2).reshape(1, 1)

    @pl.when(pl.program_id(0) == 0)
    def _init():
        loss_ref[...] = part

    @pl.when(pl.program_id(0) != 0)
    def _acc():
        loss_ref[...] += part


def _const_spec(shape):
    return pl.BlockSpec(shape, lambda i: (0,) * len(shape))


GD = 128


def _sc_gather(codebook_padded, idx):
    n = idx.shape[0]
    info = plsc.get_sparse_core_info()
    ns = info.num_subcores
    nw = info.num_cores * ns
    bpw = n // nw
    rows_per_tile = NUM_EMB // ns
    mesh = plsc.VectorSubcoreMesh(core_axis_name="c", subcore_axis_name="s")

    @functools.partial(
        pl.kernel, mesh=mesh,
        out_type=jax.ShapeDtypeStruct((n, GD), jnp.float32),
        scratch_types=[
            pltpu.VMEM((bpw,), jnp.int32),
            pltpu.VMEM((bpw, GD), jnp.float32),
            pltpu.VMEM_SHARED((NUM_EMB, GD), jnp.float32),
            pltpu.SemaphoreType.DMA,
        ],
    )
    def gather_k(table_hbm, idx_hbm, out_hbm, idx_v, rows_v, shared, sem):
        cid = lax.axis_index("c")
        sid = lax.axis_index("s")
        wid = sid * info.num_cores + cid
        sbase = sid * rows_per_tile
        pltpu.sync_copy(table_hbm.at[pl.ds(sbase, rows_per_tile)],
                        shared.at[pl.ds(sbase, rows_per_tile)])
        plsc.subcore_barrier()
        base = wid * bpw
        pltpu.sync_copy(idx_hbm.at[pl.ds(base, bpw)], idx_v)
        pltpu.async_copy(shared.at[idx_v], rows_v, sem).wait()
        pltpu.sync_copy(rows_v, out_hbm.at[pl.ds(base, bpw)])

    return gather_k(codebook_padded, idx)


def kernel(x, W1, b1, W2, b2, W3, b3, codebook, D1, c1, D2, c2, D3, c3):
    n2, cb_pad = pl.pallas_call(
        _n2_body,
        in_specs=[pl.BlockSpec((NUM_EMB, LATENT_DIM), lambda: (0, 0))],
        out_specs=[
            pl.BlockSpec((1, NUM_EMB), lambda: (0, 0)),
            pl.BlockSpec((NUM_EMB, GD), lambda: (0, 0)),
        ],
        out_shape=[
            jax.ShapeDtypeStruct((1, NUM_EMB), jnp.float32),
            jax.ShapeDtypeStruct((NUM_EMB, GD), jnp.float32),
        ],
    )(codebook)

    H = B // 2
    NBH = H // BT

    def stage_a(phase):
        off = phase * NBH
        return pl.pallas_call(
            _enc_vq_body,
            grid=(NBH,),
            in_specs=[
                pl.BlockSpec((BT, INPUT_DIM), lambda i: (i + off, 0)),
                _const_spec((INPUT_DIM, 512)),
                _const_spec((1, 512)),
                _const_spec((512, 256)),
                _const_spec((1, 256)),
                _const_spec((256, LATENT_DIM)),
                _const_spec((1, LATENT_DIM)),
                _const_spec((NUM_EMB, LATENT_DIM)),
                _const_spec((1, NUM_EMB)),
            ],
            out_specs=[
                pl.BlockSpec((BT, LATENT_DIM), lambda i: (i, 0)),
                pl.BlockSpec((1, 1, BT), lambda i: (i, 0, 0)),
            ],
            out_shape=[
                jax.ShapeDtypeStruct((H, LATENT_DIM), jnp.float32),
                jax.ShapeDtypeStruct((NBH, 1, BT), jnp.int32),
            ],
        )(x, W1, b1.reshape(1, -1), W2, b2.reshape(1, -1), W3,
          b3.reshape(1, -1), codebook, n2)

    z0, i0 = stage_a(0)
    idx0 = i0.reshape(H)
    zq0 = _sc_gather(cb_pad, idx0)
    z1, i1 = stage_a(1)
    idx1 = i1.reshape(H)
    zq1 = _sc_gather(cb_pad, idx1)

    z = jnp.concatenate([z0, z1], axis=0)
    idx = jnp.concatenate([idx0, idx1], axis=0)
    zq = jnp.concatenate([zq0, zq1], axis=0)

    xr, loss = pl.pallas_call(
        _dec_body,
        grid=(NB,),
        in_specs=[
            pl.BlockSpec((BT, LATENT_DIM), lambda i: (i, 0)),
            pl.BlockSpec((BT, GD), lambda i: (i, 0)),
            _const_spec((LATENT_DIM, 256)),
            _const_spec((1, 256)),
            _const_spec((256, 512)),
            _const_spec((1, 512)),
            _const_spec((512, INPUT_DIM)),
            _const_spec((1, INPUT_DIM)),
        ],
        out_specs=[
            pl.BlockSpec((BT, INPUT_DIM), lambda i: (i, 0)),
            _const_spec((1, 1)),
        ],
        out_shape=[
            jax.ShapeDtypeStruct((B, INPUT_DIM), jnp.float32),
            jax.ShapeDtypeStruct((1, 1), jnp.float32),
        ],
    )(z, zq, D1, c1.reshape(1, -1), D2, c2.reshape(1, -1), D3,
      c3.reshape(1, -1))

    commitment_loss = 0.25 * (loss[0, 0] / (B * LATENT_DIM))
    return (xr, z, idx, commitment_loss)


# speedup vs baseline: 1.2454x; 1.0265x over previous
import functools

import jax
import jax.numpy as jnp
from jax import lax
from jax.experimental import pallas as pl
from jax.experimental.pallas import tpu as pltpu
from jax.experimental.pallas import tpu_sc as plsc

B = 8192
INPUT_DIM = 768
LATENT_DIM = 64
NUM_EMB = 8192

BT = 512
NB = B // BT
CHUNK = 2048
NCHUNK = NUM_EMB // CHUNK


def _dot(a, b, dims):
    return lax.dot_general(a, b, (dims, ((), ())),
                           preferred_element_type=jnp.float32)


def _n2_body(cb_ref, n2_ref, cbp_ref):
    cb = cb_ref[...]
    n2_ref[...] = jnp.sum(cb * cb, axis=1).reshape(1, NUM_EMB)
    cbp_ref[:, :LATENT_DIM] = cb
    cbp_ref[:, LATENT_DIM:] = jnp.zeros((NUM_EMB, GD - LATENT_DIM), jnp.float32)


def _enc_vq_body(x_ref, W1_ref, b1_ref, W2_ref, b2_ref, W3_ref, b3_ref,
                 cb_ref, n2_ref, z_ref, idx_ref):
    x = x_ref[...]
    h = jnp.maximum(_dot(x, W1_ref[...], ((1,), (0,))) + b1_ref[...], 0.0)
    h = jnp.maximum(_dot(h, W2_ref[...], ((1,), (0,))) + b2_ref[...], 0.0)
    z = _dot(h, W3_ref[...], ((1,), (0,))) + b3_ref[...]
    z_ref[...] = z

    zz = jnp.sum(z * z, axis=1, keepdims=True)
    z2 = z + z
    best = jnp.full((BT,), jnp.inf, dtype=jnp.float32)
    besti = jnp.zeros((BT,), dtype=jnp.int32)
    for j in range(NCHUNK):
        cb = cb_ref[j * CHUNK:(j + 1) * CHUNK, :]
        n2 = n2_ref[0:1, j * CHUNK:(j + 1) * CHUNK]
        d = zz - _dot(z2, cb, ((1,), (1,))) + n2
        lmin = jnp.min(d, axis=1)
        col = lax.broadcasted_iota(jnp.int32, (BT, CHUNK), 1)
        lidx = jnp.min(jnp.where(d == lmin[:, None], col, NUM_EMB), axis=1)
        upd = lmin < best
        best = jnp.where(upd, lmin, best)
        besti = jnp.where(upd, lidx + j * CHUNK, besti)
    idx_ref[0, 0, :] = besti


def _dec_body(z_ref, zq0_ref, zq1_ref, D1_ref, c1_ref, D2_ref, c2_ref,
              D3_ref, c3_ref, xr_ref, loss_ref):
    z = z_ref[...]
    in_first_half = pl.program_id(0) < (NB // 2)
    zq = jnp.where(in_first_half, zq0_ref[:, :LATENT_DIM],
                   zq1_ref[:, :LATENT_DIM])
    zst = z + (zq - z)
    h = jnp.maximum(_dot(zst, D1_ref[...], ((1,), (0,))) + c1_ref[...], 0.0)
    h = jnp.maximum(_dot(h, D2_ref[...], ((1,), (0,))) + c2_ref[...], 0.0)
    xr_ref[...] = jnp.tanh(_dot(h, D3_ref[...], ((1,), (0,))) + c3_ref[...])

    part = jnp.sum((zq - z) ** 2).reshape(1, 1)

    @pl.when(pl.program_id(0) == 0)
    def _init():
        loss_ref[...] = part

    @pl.when(pl.program_id(0) != 0)
    def _acc():
        loss_ref[...] += part


def _const_spec(shape):
    return pl.BlockSpec(shape, lambda i: (0,) * len(shape))


GD = 128


def _sc_gather(codebook_padded, idx):
    n = idx.shape[0]
    info = plsc.get_sparse_core_info()
    ns = info.num_subcores
    nw = info.num_cores * ns
    bpw = n // nw
    rows_per_tile = NUM_EMB // ns
    mesh = plsc.VectorSubcoreMesh(core_axis_name="c", subcore_axis_name="s")

    @functools.partial(
        pl.kernel, mesh=mesh,
        out_type=jax.ShapeDtypeStruct((n, GD), jnp.float32),
        scratch_types=[
            pltpu.VMEM((bpw,), jnp.int32),
            pltpu.VMEM((bpw, GD), jnp.float32),
            pltpu.VMEM_SHARED((NUM_EMB, GD), jnp.float32),
            pltpu.SemaphoreType.DMA,
        ],
    )
    def gather_k(table_hbm, idx_hbm, out_hbm, idx_v, rows_v, shared, sem):
        cid = lax.axis_index("c")
        sid = lax.axis_index("s")
        wid = sid * info.num_cores + cid
        sbase = sid * rows_per_tile
        pltpu.sync_copy(table_hbm.at[pl.ds(sbase, rows_per_tile)],
                        shared.at[pl.ds(sbase, rows_per_tile)])
        plsc.subcore_barrier()
        base = wid * bpw
        pltpu.sync_copy(idx_hbm.at[pl.ds(base, bpw)], idx_v)
        pltpu.async_copy(shared.at[idx_v], rows_v, sem).wait()
        pltpu.sync_copy(rows_v, out_hbm.at[pl.ds(base, bpw)])

    return gather_k(codebook_padded, idx)


def kernel(x, W1, b1, W2, b2, W3, b3, codebook, D1, c1, D2, c2, D3, c3):
    n2, cb_pad = pl.pallas_call(
        _n2_body,
        in_specs=[pl.BlockSpec((NUM_EMB, LATENT_DIM), lambda: (0, 0))],
        out_specs=[
            pl.BlockSpec((1, NUM_EMB), lambda: (0, 0)),
            pl.BlockSpec((NUM_EMB, GD), lambda: (0, 0)),
        ],
        out_shape=[
            jax.ShapeDtypeStruct((1, NUM_EMB), jnp.float32),
            jax.ShapeDtypeStruct((NUM_EMB, GD), jnp.float32),
        ],
    )(codebook)

    H = B // 2
    NBH = H // BT

    def stage_a(phase):
        off = phase * NBH
        return pl.pallas_call(
            _enc_vq_body,
            grid=(NBH,),
            in_specs=[
                pl.BlockSpec((BT, INPUT_DIM), lambda i: (i + off, 0)),
                _const_spec((INPUT_DIM, 512)),
                _const_spec((1, 512)),
                _const_spec((512, 256)),
                _const_spec((1, 256)),
                _const_spec((256, LATENT_DIM)),
                _const_spec((1, LATENT_DIM)),
                _const_spec((NUM_EMB, LATENT_DIM)),
                _const_spec((1, NUM_EMB)),
            ],
            out_specs=[
                pl.BlockSpec((BT, LATENT_DIM), lambda i: (i, 0)),
                pl.BlockSpec((1, 1, BT), lambda i: (i, 0, 0)),
            ],
            out_shape=[
                jax.ShapeDtypeStruct((H, LATENT_DIM), jnp.float32),
                jax.ShapeDtypeStruct((NBH, 1, BT), jnp.int32),
            ],
        )(x, W1, b1.reshape(1, -1), W2, b2.reshape(1, -1), W3,
          b3.reshape(1, -1), codebook, n2)

    z0, i0 = stage_a(0)
    idx0 = i0.reshape(H)
    zq0 = _sc_gather(cb_pad, idx0)
    z1, i1 = stage_a(1)
    idx1 = i1.reshape(H)
    zq1 = _sc_gather(cb_pad, idx1)

    z = jnp.concatenate([z0, z1], axis=0)
    idx = jnp.concatenate([idx0, idx1], axis=0)

    xr, loss = pl.pallas_call(
        _dec_body,
        grid=(NB,),
        in_specs=[
            pl.BlockSpec((BT, LATENT_DIM), lambda i: (i, 0)),
            pl.BlockSpec((BT, GD), lambda i: (jnp.minimum(i, NB // 2 - 1), 0)),
            pl.BlockSpec((BT, GD), lambda i: (jnp.maximum(i - NB // 2, 0), 0)),
            _const_spec((LATENT_DIM, 256)),
            _const_spec((1, 256)),
            _const_spec((256, 512)),
            _const_spec((1, 512)),
            _const_spec((512, INPUT_DIM)),
            _const_spec((1, INPUT_DIM)),
        ],
        out_specs=[
            pl.BlockSpec((BT, INPUT_DIM), lambda i: (i, 0)),
            _const_spec((1, 1)),
        ],
        out_shape=[
            jax.ShapeDtypeStruct((B, INPUT_DIM), jnp.float32),
            jax.ShapeDtypeStruct((1, 1), jnp.float32),
        ],
    )(z, zq0, zq1, D1, c1.reshape(1, -1), D2, c2.reshape(1, -1), D3,
      c3.reshape(1, -1))

    commitment_loss = 0.25 * (loss[0, 0] / (B * LATENT_DIM))
    return (xr, z, idx, commitment_loss)
